# baseline (speedup 1.0000x reference)
import functools
import jax
import jax.numpy as jnp
from jax import lax
from jax.experimental import pallas as pl
from jax.experimental.pallas import tpu as pltpu
from jax.experimental.pallas import tpu_sc as plsc

_HN = 8
_P = 5
_X, _Y, _Z = 64, 64, 8
_S = _X * _Y * _Z
_NPAIR = _P * 4
_NC, _NS, _L = 2, 16, 16


def _rne_bf16_bits(a):
    b = lax.bitcast_convert_type(a, jnp.int32)
    return b + 32767 + (lax.shift_right_logical(b, 16) & 1)


def _vproj_body(wv_ref, bv_ref, val_ref, out_ref):
    v = val_ref[0]
    x = lax.dot_general(
        wv_ref[...], v, (((1,), (0,)), ((), ())),
        preferred_element_type=jnp.float32) + bv_ref[...]
    xs = jnp.concatenate([x[:, -1:], x[:, :-1]], axis=-1)
    out_ref[0] = (_rne_bf16_bits(xs) & (-65536)) | lax.shift_right_logical(
        _rne_bf16_bits(x), 16)


def _vproj(Wv, bv, value2, sb=4096):
    bs, C, S = value2.shape
    return pl.pallas_call(
        _vproj_body,
        grid=(bs, S // sb),
        in_specs=[
            pl.BlockSpec((C, C), lambda b, s: (0, 0)),
            pl.BlockSpec((C, 1), lambda b, s: (0, 0)),
            pl.BlockSpec((1, C, sb), lambda b, s: (b, 0, s)),
        ],
        out_specs=pl.BlockSpec((1, C, sb), lambda b, s: (b, 0, s)),
        out_shape=jax.ShapeDtypeStruct((bs, C, S), jnp.int32),
    )(Wv, bv.reshape(C, 1), value2)


def _samp_body(q_ref, rp_ref, wox_ref, woy_ref, woz_ref,
               box_ref, boy_ref, boz_ref, wa_ref, ba_ref,
               idx_ref, wp_ref):
    q = q_ref[0]
    cdims = (((0,), (1,)), ((), ()))
    ox = lax.dot_general(wox_ref[...], q, cdims,
                         preferred_element_type=jnp.float32) + box_ref[...]
    oy = lax.dot_general(woy_ref[...], q, cdims,
                         preferred_element_type=jnp.float32) + boy_ref[...]
    oz = lax.dot_general(woz_ref[...], q, cdims,
                         preferred_element_type=jnp.float32) + boz_ref[...]
    al = lax.dot_general(wa_ref[...], q, cdims,
                         preferred_element_type=jnp.float32) + ba_ref[...]
    qb = q.shape[0]
    al = al.reshape(_HN, _P, qb)
    al = al - jnp.max(al, axis=1, keepdims=True)
    e = jnp.exp(al)
    aw = e / jnp.sum(e, axis=1, keepdims=True)

    rx = rp_ref[0, 0:1, :].reshape(1, 1, 1, qb)
    ry = rp_ref[0, 1:2, :].reshape(1, 1, 1, qb)
    rz = rp_ref[0, 2:3, :].reshape(1, 1, 1, qb)
    u = rx * float(_X) + ox.reshape(_HN, _P, 1, qb) - 0.5
    v = ry * float(_Y) + oy.reshape(_HN, _P, 1, qb) - 0.5
    t = rz * float(_Z) + oz.reshape(_HN, _P, 1, qb) - 0.5

    cnr = lax.broadcasted_iota(jnp.int32, (1, 1, 4, 1), 2)
    du = (cnr // 2).astype(jnp.float32)
    dv = (cnr % 2).astype(jnp.float32)

    def corner(coord, dfrac, hi):
        f = jnp.floor(coord)
        r = coord - f
        ci = f + dfrac
        wgt = dfrac * r + (1.0 - dfrac) * (1.0 - r)
        ok = (ci >= 0.0) & (ci <= float(hi))
        cl = jnp.clip(ci, 0.0, float(hi)).astype(jnp.int32)
        return cl, wgt, ok

    ui, wu, mu = corner(u, du, _X - 1)
    vi, wv, mv = corner(v, dv, _Y - 1)
    kcol = ui * (_Y * _Z) + vi * _Z
    wxy = wu * wv * (mu & mv).astype(jnp.float32) * aw.reshape(_HN, _P, 1, qb)

    tb = jnp.floor(t)
    rt = t - tb
    mz0 = ((tb >= 0.0) & (tb <= float(_Z - 1))).astype(jnp.float32)
    mz1 = ((tb >= -1.0) & (tb <= float(_Z - 2))).astype(jnp.float32)
    wz0 = (1.0 - rt) * mz0
    wz1 = rt * mz1
    tbc = jnp.clip(tb, -1.0, float(_Z - 1))
    swap = tbc == float(_Z - 1)
    kz = jnp.where(swap, float(_Z - 1), tbc + 1.0).astype(jnp.int32)
    wza = jnp.where(swap, 0.0, wz0)
    wzb = jnp.where(swap, wz0, wz1)

    idx_ref[...] = kcol + kz
    wa = wxy * wza
    wb = wxy * wzb
    wp_ref[...] = (_rne_bf16_bits(wa) & (-65536)) | lax.shift_right_logical(
        _rne_bf16_bits(wb), 16)


def _sampling(query, rpT, Wox, Woy, Woz, box, boy, boz, Wa, ba, qb=512):
    bs, Nq, C = query.shape
    hp = _HN * _P
    grid = (bs, Nq // qb)
    wspec = pl.BlockSpec((C, hp), lambda b, s: (0, 0))
    bspec = pl.BlockSpec((hp, 1), lambda b, s: (0, 0))
    ospec = pl.BlockSpec((_HN, _P, 4, qb), lambda b, s: (b, 0, 0, s))
    oshape = jax.ShapeDtypeStruct((bs * _HN, _P, 4, Nq), jnp.int32)
    return pl.pallas_call(
        _samp_body,
        grid=grid,
        in_specs=[
            pl.BlockSpec((1, qb, C), lambda b, s: (b, s, 0)),
            pl.BlockSpec((1, 3, qb), lambda b, s: (b, 0, s)),
            wspec, wspec, wspec, bspec, bspec, bspec, wspec, bspec,
        ],
        out_specs=[ospec, ospec],
        out_shape=[oshape, oshape],
    )(query, rpT, Wox, Woy, Woz,
      box.reshape(hp, 1), boy.reshape(hp, 1), boz.reshape(hp, 1),
      Wa, ba.reshape(hp, 1))


def _sc_body(v_hbm, idx_hbm, wp_hbm, out_hbm,
             t0, t1, t2, idxb, wpb, outb, sem0, sem1, tsem):
    nbh = v_hbm.shape[0]
    ntt = nbh // 2
    wid = lax.axis_index("s") * _NC + lax.axis_index("c")
    c0 = (wid % 16) * 2
    bh_base = (wid // 16) * ntt

    def issue(bh, qs, buf, sem):
        pltpu.async_copy(idx_hbm.at[bh, :, pl.ds(qs, 256)], idxb.at[buf], sem)
        pltpu.async_copy(wp_hbm.at[bh, :, pl.ds(qs, 256)], wpb.at[buf], sem)

    def drain(bh, qs, buf, sem):
        pltpu.make_async_copy(idx_hbm.at[bh, :, pl.ds(qs, 256)],
                              idxb.at[buf], sem).wait()
        pltpu.make_async_copy(wp_hbm.at[bh, :, pl.ds(qs, 256)],
                              wpb.at[buf], sem).wait()

    tabs = (t0, t1, t2)

    def make_compute(s0, s1):
        def compute_slice(qs, buf):
            def qc_step(qc, __):
                o = qc * _L
                acc0 = jnp.zeros((_L,), jnp.float32)
                acc1 = jnp.zeros((_L,), jnp.float32)
                for j in range(_NPAIR):
                    iv = idxb[buf, j, pl.ds(o, _L)]
                    wpk = wpb[buf, j, pl.ds(o, _L)]
                    wa = plsc.bitcast(wpk & (-65536), jnp.float32)
                    wb = plsc.bitcast(wpk << 16, jnp.float32)
                    g0 = plsc.load_gather(tabs[s0], [iv])
                    g1 = plsc.load_gather(tabs[s1], [iv])
                    hi0 = plsc.bitcast(g0 & (-65536), jnp.float32)
                    lo0 = plsc.bitcast(g0 << 16, jnp.float32)
                    hi1 = plsc.bitcast(g1 & (-65536), jnp.float32)
                    lo1 = plsc.bitcast(g1 << 16, jnp.float32)
                    acc0 = acc0 + wa * hi0 + wb * lo0
                    acc1 = acc1 + wa * hi1 + wb * lo1
                outb[0, pl.ds(qs + o, _L)] = acc0
                outb[1, pl.ds(qs + o, _L)] = acc1
                return __

            lax.fori_loop(0, 16, qc_step, 0)

        return compute_slice

    pltpu.async_copy(v_hbm.at[bh_base, c0], t0, tsem)

    for tt in range(ntt):
        bh = bh_base + tt
        s0 = (2 * tt) % 3
        s1 = (2 * tt + 1) % 3
        sp = (2 * tt + 2) % 3
        pltpu.make_async_copy(v_hbm.at[bh, c0], tabs[s0], tsem).wait()
        pltpu.sync_copy(v_hbm.at[bh, c0 + 1], tabs[s1])
        if tt < ntt - 1:
            pltpu.async_copy(v_hbm.at[bh + 1, c0], tabs[sp], tsem)
        compute_slice = make_compute(s0, s1)
        issue(bh, 0, 0, sem0)

        def qp_step(qp, _):
            qs = qp * 512
            drain(bh, qs, 0, sem0)
            issue(bh, qs + 256, 1, sem1)
            compute_slice(qs, 0)
            drain(bh, qs + 256, 1, sem1)

            @pl.when(qp < 7)
            def _prefetch():
                issue(bh, qs + 512, 0, sem0)

            compute_slice(qs + 256, 1)
            return _

        lax.fori_loop(0, 8, qp_step, 0)
        pltpu.sync_copy(outb, out_hbm.at[bh, pl.ds(c0, 2)])


def _sc_gather(v_t, idx, wgt_p):
    nbh, _, Nq = idx.shape
    kfn = pl.kernel(
        _sc_body,
        out_type=jax.ShapeDtypeStruct((nbh, 32, Nq), jnp.float32),
        mesh=plsc.VectorSubcoreMesh(core_axis_name="c", subcore_axis_name="s",
                                    num_cores=_NC, num_subcores=_NS),
        scratch_types=[
            pltpu.VMEM((_S,), jnp.int32),
            pltpu.VMEM((_S,), jnp.int32),
            pltpu.VMEM((_S,), jnp.int32),
            pltpu.VMEM((2, _NPAIR, 256), jnp.int32),
            pltpu.VMEM((2, _NPAIR, 256), jnp.int32),
            pltpu.VMEM((2, Nq), jnp.float32),
            pltpu.SemaphoreType.DMA,
            pltpu.SemaphoreType.DMA,
            pltpu.SemaphoreType.DMA,
        ],
        compiler_params=pltpu.CompilerParams(needs_layout_passes=False),
    )
    return kfn(v_t, idx, wgt_p)


def _oproj_body(s_ref, wo_ref, bo_ref, out_ref):
    s = s_ref[0]
    out_ref[0] = lax.dot_general(
        s, wo_ref[...], (((0,), (0,)), ((), ())),
        preferred_element_type=jnp.float32) + bo_ref[...]


def _oproj(samp2, Wout, bout, qb=1024):
    bs, C, Nq = samp2.shape
    return pl.pallas_call(
        _oproj_body,
        grid=(bs, Nq // qb),
        in_specs=[
            pl.BlockSpec((1, C, qb), lambda b, s: (b, 0, s)),
            pl.BlockSpec((C, C), lambda b, s: (0, 0)),
            pl.BlockSpec((1, C), lambda b, s: (0, 0)),
        ],
        out_specs=pl.BlockSpec((1, qb, C), lambda b, s: (b, s, 0)),
        out_shape=jax.ShapeDtypeStruct((bs, Nq, C), jnp.float32),
    )(samp2, Wout, bout.reshape(1, C))


@jax.jit
def kernel(query, value, reference_points, Wv, bv, Woff, boff, Wa, ba,
           Wout, bout):
    bs, Nq, C = query.shape
    ch = C // _HN

    value2 = value.reshape(bs, C, _S)
    W3 = Woff.reshape(C, _HN * _P, 3)
    b3 = boff.reshape(_HN * _P, 3)
    rpT = jnp.transpose(reference_points, (0, 2, 1))

    outs = []
    for b in range(bs):
        v_t = _vproj(Wv, bv, value2[b:b + 1]).reshape(_HN, ch, _S)
        idx, wgp = _sampling(query[b:b + 1], rpT[b:b + 1],
                             W3[:, :, 0], W3[:, :, 1], W3[:, :, 2],
                             b3[:, 0], b3[:, 1], b3[:, 2], Wa, ba)
        idx = idx.reshape(_HN, _NPAIR, Nq)
        wgp = wgp.reshape(_HN, _NPAIR, Nq)
        samp = _sc_gather(v_t, idx, wgp)
        outs.append(_oproj(samp.reshape(1, C, Nq), Wout, bout))
    return jnp.concatenate(outs, axis=0)


# speedup vs baseline: 1.0390x; 1.0390x over previous
import functools
import jax
import jax.numpy as jnp
from jax import lax
from jax.experimental import pallas as pl
from jax.experimental.pallas import tpu as pltpu
from jax.experimental.pallas import tpu_sc as plsc

_HN = 8
_P = 5
_X, _Y, _Z = 64, 64, 8
_S = _X * _Y * _Z
_NPAIR = _P * 4
_NC, _NS, _L = 2, 16, 16


def _rne_bf16_bits(a):
    b = lax.bitcast_convert_type(a, jnp.int32)
    return b + 32767 + (lax.shift_right_logical(b, 16) & 1)


def _vproj_body(wv_ref, bv_ref, val_ref, out_ref):
    v = val_ref[0]
    x = lax.dot_general(
        wv_ref[...], v, (((1,), (0,)), ((), ())),
        preferred_element_type=jnp.float32) + bv_ref[...]
    xs = jnp.concatenate([x[:, -1:], x[:, :-1]], axis=-1)
    out_ref[0] = (_rne_bf16_bits(xs) & (-65536)) | lax.shift_right_logical(
        _rne_bf16_bits(x), 16)


def _vproj(Wv, bv, value2, sb=4096):
    bs, C, S = value2.shape
    return pl.pallas_call(
        _vproj_body,
        grid=(bs, S // sb),
        in_specs=[
            pl.BlockSpec((C, C), lambda b, s: (0, 0)),
            pl.BlockSpec((C, 1), lambda b, s: (0, 0)),
            pl.BlockSpec((1, C, sb), lambda b, s: (b, 0, s)),
        ],
        out_specs=pl.BlockSpec((1, C, sb), lambda b, s: (b, 0, s)),
        out_shape=jax.ShapeDtypeStruct((bs, C, S), jnp.int32),
    )(Wv, bv.reshape(C, 1), value2)


def _samp_body(q_ref, rp_ref, wox_ref, woy_ref, woz_ref,
               box_ref, boy_ref, boz_ref, wa_ref, ba_ref,
               idx_ref, wp_ref):
    q = q_ref[0]
    cdims = (((0,), (1,)), ((), ()))
    ox = lax.dot_general(wox_ref[...], q, cdims,
                         preferred_element_type=jnp.float32) + box_ref[...]
    oy = lax.dot_general(woy_ref[...], q, cdims,
                         preferred_element_type=jnp.float32) + boy_ref[...]
    oz = lax.dot_general(woz_ref[...], q, cdims,
                         preferred_element_type=jnp.float32) + boz_ref[...]
    al = lax.dot_general(wa_ref[...], q, cdims,
                         preferred_element_type=jnp.float32) + ba_ref[...]
    qb = q.shape[0]
    al = al.reshape(_HN, _P, qb)
    al = al - jnp.max(al, axis=1, keepdims=True)
    e = jnp.exp(al)
    aw = e / jnp.sum(e, axis=1, keepdims=True)

    rx = rp_ref[0, 0:1, :].reshape(1, 1, 1, qb)
    ry = rp_ref[0, 1:2, :].reshape(1, 1, 1, qb)
    rz = rp_ref[0, 2:3, :].reshape(1, 1, 1, qb)
    u = rx * float(_X) + ox.reshape(_HN, _P, 1, qb) - 0.5
    v = ry * float(_Y) + oy.reshape(_HN, _P, 1, qb) - 0.5
    t = rz * float(_Z) + oz.reshape(_HN, _P, 1, qb) - 0.5

    cnr = lax.broadcasted_iota(jnp.int32, (1, 1, 4, 1), 2)
    du = (cnr // 2).astype(jnp.float32)
    dv = (cnr % 2).astype(jnp.float32)

    def corner(coord, dfrac, hi):
        f = jnp.floor(coord)
        r = coord - f
        ci = f + dfrac
        wgt = dfrac * r + (1.0 - dfrac) * (1.0 - r)
        ok = (ci >= 0.0) & (ci <= float(hi))
        cl = jnp.clip(ci, 0.0, float(hi)).astype(jnp.int32)
        return cl, wgt, ok

    ui, wu, mu = corner(u, du, _X - 1)
    vi, wv, mv = corner(v, dv, _Y - 1)
    kcol = ui * (_Y * _Z) + vi * _Z
    wxy = wu * wv * (mu & mv).astype(jnp.float32) * aw.reshape(_HN, _P, 1, qb)

    tb = jnp.floor(t)
    rt = t - tb
    mz0 = ((tb >= 0.0) & (tb <= float(_Z - 1))).astype(jnp.float32)
    mz1 = ((tb >= -1.0) & (tb <= float(_Z - 2))).astype(jnp.float32)
    wz0 = (1.0 - rt) * mz0
    wz1 = rt * mz1
    tbc = jnp.clip(tb, -1.0, float(_Z - 1))
    swap = tbc == float(_Z - 1)
    kz = jnp.where(swap, float(_Z - 1), tbc + 1.0).astype(jnp.int32)
    wza = jnp.where(swap, 0.0, wz0)
    wzb = jnp.where(swap, wz0, wz1)

    ip = kcol + kz
    g0 = ip[..., 0:128] | (ip[..., 128:256] << 16)
    g1 = ip[..., 256:384] | (ip[..., 384:512] << 16)
    idx_ref[...] = jnp.concatenate([g0, g1], axis=-1)
    wa = wxy * wza
    wb = wxy * wzb
    wp_ref[...] = (_rne_bf16_bits(wa) & (-65536)) | lax.shift_right_logical(
        _rne_bf16_bits(wb), 16)


def _sampling(query, rpT, Wox, Woy, Woz, box, boy, boz, Wa, ba, qb=512):
    bs, Nq, C = query.shape
    hp = _HN * _P
    grid = (bs, Nq // qb)
    wspec = pl.BlockSpec((C, hp), lambda b, s: (0, 0))
    bspec = pl.BlockSpec((hp, 1), lambda b, s: (0, 0))
    ispec = pl.BlockSpec((_HN, _P, 4, qb // 2), lambda b, s: (b, 0, 0, s))
    ishape = jax.ShapeDtypeStruct((bs * _HN, _P, 4, Nq // 2), jnp.int32)
    ospec = pl.BlockSpec((_HN, _P, 4, qb), lambda b, s: (b, 0, 0, s))
    oshape = jax.ShapeDtypeStruct((bs * _HN, _P, 4, Nq), jnp.int32)
    return pl.pallas_call(
        _samp_body,
        grid=grid,
        in_specs=[
            pl.BlockSpec((1, qb, C), lambda b, s: (b, s, 0)),
            pl.BlockSpec((1, 3, qb), lambda b, s: (b, 0, s)),
            wspec, wspec, wspec, bspec, bspec, bspec, wspec, bspec,
        ],
        out_specs=[ispec, ospec],
        out_shape=[ishape, oshape],
    )(query, rpT, Wox, Woy, Woz,
      box.reshape(hp, 1), boy.reshape(hp, 1), boz.reshape(hp, 1),
      Wa, ba.reshape(hp, 1))


def _sc_body(v_hbm, idx_hbm, wp_hbm, out_hbm,
             t0, t1, t2, idxb, wpb, outb, sem0, sem1, tsem):
    wid = lax.axis_index("s") * _NC + lax.axis_index("c")
    c0 = (wid % 16) * 2
    bh_base = (wid // 16) * 8

    def issue(bh, si, buf, sem):
        pltpu.async_copy(idx_hbm.at[bh, :, si], idxb.at[buf, :, pl.ds(0, 128)], sem)
        pltpu.async_copy(wp_hbm.at[bh, :, pl.ds(si * 256, 256)],
                         wpb.at[buf], sem)

    def drain(bh, si, buf, sem):
        pltpu.make_async_copy(idx_hbm.at[bh, :, si],
                              idxb.at[buf, :, pl.ds(0, 128)], sem).wait()
        pltpu.make_async_copy(wp_hbm.at[bh, :, pl.ds(si * 256, 256)],
                              wpb.at[buf], sem).wait()

    tabs = (t0, t1, t2)

    def make_compute(s0, s1):
        def compute_slice(qs, buf):
            def qc_step(qc, __):
                o = qc * _L
                a0A = jnp.zeros((_L,), jnp.float32)
                a1A = jnp.zeros((_L,), jnp.float32)
                a0B = jnp.zeros((_L,), jnp.float32)
                a1B = jnp.zeros((_L,), jnp.float32)
                for j in range(_NPAIR):
                    ivp = idxb[buf, j, pl.ds(o, _L)]
                    ivA = ivp & 65535
                    ivB = lax.shift_right_logical(ivp, 16)
                    wpkA = wpb[buf, j, pl.ds(o, _L)]
                    wpkB = wpb[buf, j, pl.ds(128 + o, _L)]
                    waA = plsc.bitcast(wpkA & (-65536), jnp.float32)
                    wbA = plsc.bitcast(wpkA << 16, jnp.float32)
                    waB = plsc.bitcast(wpkB & (-65536), jnp.float32)
                    wbB = plsc.bitcast(wpkB << 16, jnp.float32)
                    g0A = plsc.load_gather(tabs[s0], [ivA])
                    g1A = plsc.load_gather(tabs[s1], [ivA])
                    g0B = plsc.load_gather(tabs[s0], [ivB])
                    g1B = plsc.load_gather(tabs[s1], [ivB])
                    a0A = a0A + waA * plsc.bitcast(g0A & (-65536), jnp.float32) \
                              + wbA * plsc.bitcast(g0A << 16, jnp.float32)
                    a1A = a1A + waA * plsc.bitcast(g1A & (-65536), jnp.float32) \
                              + wbA * plsc.bitcast(g1A << 16, jnp.float32)
                    a0B = a0B + waB * plsc.bitcast(g0B & (-65536), jnp.float32) \
                              + wbB * plsc.bitcast(g0B << 16, jnp.float32)
                    a1B = a1B + waB * plsc.bitcast(g1B & (-65536), jnp.float32) \
                              + wbB * plsc.bitcast(g1B << 16, jnp.float32)
                outb[0, pl.ds(qs + o, _L)] = a0A
                outb[1, pl.ds(qs + o, _L)] = a1A
                outb[0, pl.ds(qs + 128 + o, _L)] = a0B
                outb[1, pl.ds(qs + 128 + o, _L)] = a1B
                return __

            lax.fori_loop(0, 8, qc_step, 0)

        return compute_slice

    pltpu.async_copy(v_hbm.at[bh_base, c0], t0, tsem)

    for tt in range(8):
        bh = bh_base + tt
        s0 = (2 * tt) % 3
        s1 = (2 * tt + 1) % 3
        sp = (2 * tt + 2) % 3
        pltpu.make_async_copy(v_hbm.at[bh, c0], tabs[s0], tsem).wait()
        pltpu.sync_copy(v_hbm.at[bh, c0 + 1], tabs[s1])
        if tt < 7:
            pltpu.async_copy(v_hbm.at[bh + 1, c0], tabs[sp], tsem)
        compute_slice = make_compute(s0, s1)
        issue(bh, 0, 0, sem0)

        def qp_step(qp, _):
            si = qp * 2
            qs = qp * 512
            drain(bh, si, 0, sem0)
            issue(bh, si + 1, 1, sem1)
            compute_slice(qs, 0)
            drain(bh, si + 1, 1, sem1)

            @pl.when(qp < 7)
            def _prefetch():
                issue(bh, si + 2, 0, sem0)

            compute_slice(qs + 256, 1)
            return _

        lax.fori_loop(0, 8, qp_step, 0)
        pltpu.sync_copy(outb, out_hbm.at[bh, pl.ds(c0, 2)])


def _sc_gather(v_t, idx, wgt_p):
    Nq = wgt_p.shape[2]
    kfn = pl.kernel(
        _sc_body,
        out_type=jax.ShapeDtypeStruct((2 * _HN, 32, Nq), jnp.float32),
        mesh=plsc.VectorSubcoreMesh(core_axis_name="c", subcore_axis_name="s",
                                    num_cores=_NC, num_subcores=_NS),
        scratch_types=[
            pltpu.VMEM((_S,), jnp.int32),
            pltpu.VMEM((_S,), jnp.int32),
            pltpu.VMEM((_S,), jnp.int32),
            pltpu.VMEM((2, _NPAIR, 256), jnp.int32),
            pltpu.VMEM((2, _NPAIR, 256), jnp.int32),
            pltpu.VMEM((2, Nq), jnp.float32),
            pltpu.SemaphoreType.DMA,
            pltpu.SemaphoreType.DMA,
            pltpu.SemaphoreType.DMA,
        ],
        compiler_params=pltpu.CompilerParams(needs_layout_passes=False),
    )
    return kfn(v_t, idx, wgt_p)


def _oproj_body(s_ref, wo_ref, bo_ref, out_ref):
    s = s_ref[0]
    out_ref[0] = lax.dot_general(
        s, wo_ref[...], (((0,), (0,)), ((), ())),
        preferred_element_type=jnp.float32) + bo_ref[...]


def _oproj(samp2, Wout, bout, qb=1024):
    bs, C, Nq = samp2.shape
    return pl.pallas_call(
        _oproj_body,
        grid=(bs, Nq // qb),
        in_specs=[
            pl.BlockSpec((1, C, qb), lambda b, s: (b, 0, s)),
            pl.BlockSpec((C, C), lambda b, s: (0, 0)),
            pl.BlockSpec((1, C), lambda b, s: (0, 0)),
        ],
        out_specs=pl.BlockSpec((1, qb, C), lambda b, s: (b, s, 0)),
        out_shape=jax.ShapeDtypeStruct((bs, Nq, C), jnp.float32),
    )(samp2, Wout, bout.reshape(1, C))


@jax.jit
def kernel(query, value, reference_points, Wv, bv, Woff, boff, Wa, ba,
           Wout, bout):
    bs, Nq, C = query.shape
    ch = C // _HN

    value2 = value.reshape(bs, C, _S)
    v_t = _vproj(Wv, bv, value2).reshape(bs * _HN, ch, _S)

    W3 = Woff.reshape(C, _HN * _P, 3)
    b3 = boff.reshape(_HN * _P, 3)
    rpT = jnp.transpose(reference_points, (0, 2, 1))
    idx, wgp = _sampling(query, rpT,
                         W3[:, :, 0], W3[:, :, 1], W3[:, :, 2],
                         b3[:, 0], b3[:, 1], b3[:, 2], Wa, ba)
    idx = idx.reshape(bs * _HN, _NPAIR, Nq // 256, 128)
    wgp = wgp.reshape(bs * _HN, _NPAIR, Nq)

    samp = _sc_gather(v_t, idx, wgp)
    samp2 = samp.reshape(bs, C, Nq)
    return _oproj(samp2, Wout, bout)
